# Initial kernel scaffold; baseline (speedup 1.0000x reference)
import jax
import jax.numpy as jnp
from jax.experimental import pallas as pl


def kernel(x, connections1, connections2, w1, b1, w2, b2):
    raise NotImplementedError("write your pallas kernel here")


# speedup vs baseline: 1.0335x; 1.0335x over previous
import functools

import jax
import jax.numpy as jnp
from jax import lax
from jax.experimental import pallas as pl
from jax.experimental.pallas import tpu as pltpu
from jax.experimental.pallas import tpu_sc as plsc

B, IN, HID, OUT = 256, 16384, 4096, 64
NNZ1, NNZ2 = 131072, 16384
NC, NS, L = 2, 16, 16
NB = B // L
MC = 2048
GB = 128


def _sc_sparse_layer(nnz, n_in, n_out):
    per_core = nnz // NC
    n_mchunks = per_core // MC
    mesh = plsc.VectorSubcoreMesh(core_axis_name="c", subcore_axis_name="s")

    @functools.partial(
        pl.kernel,
        out_type=jax.ShapeDtypeStruct((NC, NB, n_out, L), jnp.float32),
        mesh=mesh,
        compiler_params=pltpu.CompilerParams(use_tc_tiling_on_sc=False),
        scratch_types=[
            pltpu.VMEM((MC,), jnp.int32),
            pltpu.VMEM((MC,), jnp.int32),
            pltpu.VMEM((MC,), jnp.float32),
            pltpu.VMEM((MC, L), jnp.float32),
            pltpu.VMEM((n_out, L), jnp.float32),
            pltpu.SemaphoreType.DMA,
        ],
    )
    def layer(xb, out_idx, in_idx, w, out_hbm, mo_v, mi_v, mw_v, rows_v,
              acc_v, sem):
        c = lax.axis_index("c")
        bb = lax.axis_index("s")

        def zrow(r, carry):
            acc_v[r, pl.ds(0, L)] = jnp.zeros((L,), jnp.float32)
            return carry
        lax.fori_loop(0, n_out, zrow, 0)

        base_row = bb * n_in

        def mchunk(m, carry):
            e0 = c * per_core + m * MC
            pltpu.sync_copy(out_idx.at[pl.ds(e0, MC)], mo_v)
            pltpu.sync_copy(in_idx.at[pl.ds(e0, MC)], mi_v)
            pltpu.sync_copy(w.at[pl.ds(e0, MC)], mw_v)

            def bias_grp(g, carry2):
                sl = pl.ds(g * L, L)
                mi_v[sl] = mi_v[sl] + base_row
                return carry2
            lax.fori_loop(0, MC // L, bias_grp, 0)

            def gather_blk(gi, carry2):
                pltpu.async_copy(
                    xb.at[mi_v.at[pl.ds(gi * GB, GB)]],
                    rows_v.at[pl.ds(gi * GB, GB)], sem).wait()
                return carry2
            lax.fori_loop(0, MC // GB, gather_blk, 0)

            def acc_grp(g, carry2):
                e0g = g * L
                o16 = mo_v[pl.ds(e0g, L)]
                w16 = mw_v[pl.ds(e0g, L)]
                for e16 in range(L):
                    o = o16[e16]
                    wv = w16[e16]
                    acc_v[o, pl.ds(0, L)] = (
                        acc_v[o, pl.ds(0, L)]
                        + rows_v[e0g + e16, pl.ds(0, L)] * wv)
                return carry2
            lax.fori_loop(0, MC // L, acc_grp, 0)
            return carry
        lax.fori_loop(0, n_mchunks, mchunk, 0)

        pltpu.sync_copy(acc_v, out_hbm.at[c, bb])

    return layer


_layer1 = _sc_sparse_layer(NNZ1, IN, HID)
_layer2 = _sc_sparse_layer(NNZ2, HID, OUT)


def _combine_sigmoid(parts, b):
    n = parts.shape[2]
    blk = 512

    def body(p_ref, b_ref, o_ref):
        o_ref[...] = jax.nn.sigmoid(p_ref[0] + p_ref[1] + b_ref[0][..., None])

    return pl.pallas_call(
        body,
        grid=(NB, n // blk),
        in_specs=[
            pl.BlockSpec((2, 1, blk, L), lambda i, j: (0, i, j, 0)),
            pl.BlockSpec((1, blk), lambda i, j: (0, j)),
        ],
        out_specs=pl.BlockSpec((1, blk, L), lambda i, j: (i, j, 0)),
        out_shape=jax.ShapeDtypeStruct((NB, n, L), jnp.float32),
    )(parts, b.reshape(1, -1)).reshape(NB * n, L)


def _finish(parts, b):
    def body(p_ref, b_ref, o_ref):
        o_ref[...] = p_ref[0] + p_ref[1] + b_ref[0][None, :, None]

    return pl.pallas_call(
        body,
        out_shape=jax.ShapeDtypeStruct((NB, OUT, L), jnp.float32),
    )(parts, b.reshape(1, -1))


@jax.jit
def kernel(x, connections1, connections2, w1, b1, w2, b2):
    xb = x.reshape(NB, L, IN).transpose(0, 2, 1).reshape(NB * IN, L)
    h_parts = _layer1(xb, connections1[0], connections1[1], w1)
    hb = _combine_sigmoid(h_parts, b1)
    o_parts = _layer2(hb, connections2[0], connections2[1], w2)
    ob = _finish(o_parts, b2)
    return ob.transpose(0, 2, 1).reshape(B, OUT)


# speedup vs baseline: 1.4419x; 1.3951x over previous
import functools

import jax
import jax.numpy as jnp
from jax import lax
from jax.experimental import pallas as pl
from jax.experimental.pallas import tpu as pltpu
from jax.experimental.pallas import tpu_sc as plsc

B, IN, HID, OUT = 256, 16384, 4096, 64
NNZ1, NNZ2 = 131072, 16384
NC, NS, L = 2, 16, 16
NB = B // L
MC = 1024
GB = 128


def _sc_sparse_layer(nnz, n_in, n_out):
    per_core = nnz // NC
    n_mchunks = per_core // MC
    mesh = plsc.VectorSubcoreMesh(core_axis_name="c", subcore_axis_name="s")

    @functools.partial(
        pl.kernel,
        out_type=jax.ShapeDtypeStruct((NC, NB, n_out, L), jnp.float32),
        mesh=mesh,
        compiler_params=pltpu.CompilerParams(use_tc_tiling_on_sc=False),
        scratch_types=[
            pltpu.VMEM((2, MC), jnp.int32),
            pltpu.VMEM((2, MC), jnp.int32),
            pltpu.VMEM((2, MC), jnp.float32),
            pltpu.VMEM((2, MC, L), jnp.float32),
            pltpu.VMEM((n_out, L), jnp.float32),
            pltpu.SemaphoreType.DMA,
            pltpu.SemaphoreType.DMA,
        ],
    )
    def layer(xb, out_idx, in_idx, w, out_hbm, mo_v, mi_v, mw_v, rows_v,
              acc_v, sem0, sem1):
        c = lax.axis_index("c")
        bb = lax.axis_index("s")
        sems = (sem0, sem1)

        def zrow(r, carry):
            acc_v[r, pl.ds(0, L)] = jnp.zeros((L,), jnp.float32)
            return carry
        lax.fori_loop(0, n_out, zrow, 0)

        base_row = bb * n_in

        def issue(m, p):
            e0 = c * per_core + m * MC
            pltpu.sync_copy(out_idx.at[pl.ds(e0, MC)], mo_v.at[p])
            pltpu.sync_copy(in_idx.at[pl.ds(e0, MC)], mi_v.at[p])
            pltpu.sync_copy(w.at[pl.ds(e0, MC)], mw_v.at[p])

            def bias_grp(g, carry2):
                sl = pl.ds(g * L, L)
                mi_v[p, sl] = mi_v[p, sl] + base_row
                return carry2
            lax.fori_loop(0, MC // L, bias_grp, 0)
            for gi in range(MC // GB):
                pltpu.async_copy(
                    xb.at[mi_v.at[p].at[pl.ds(gi * GB, GB)]],
                    rows_v.at[p].at[pl.ds(gi * GB, GB)], sems[p])

        def drain_acc(p):
            for gi in range(MC // GB):
                pltpu.make_async_copy(
                    xb.at[mi_v.at[p].at[pl.ds(gi * GB, GB)]],
                    rows_v.at[p].at[pl.ds(gi * GB, GB)], sems[p]).wait()

            def acc_grp(g, carry2):
                e0g = g * L
                o16 = mo_v[p, pl.ds(e0g, L)]
                w16 = mw_v[p, pl.ds(e0g, L)]
                for e16 in range(L):
                    o = o16[e16]
                    wv = w16[e16]
                    acc_v[o, pl.ds(0, L)] = (
                        acc_v[o, pl.ds(0, L)]
                        + rows_v[p, e0g + e16, pl.ds(0, L)] * wv)
                return carry2
            lax.fori_loop(0, MC // L, acc_grp, 0)

        issue(0, 0)

        def mchunk(m, carry):
            @pl.when(lax.rem(m, 2) == 0)
            def _():
                @pl.when(m + 1 < n_mchunks)
                def _():
                    issue(m + 1, 1)
                drain_acc(0)

            @pl.when(lax.rem(m, 2) == 1)
            def _():
                @pl.when(m + 1 < n_mchunks)
                def _():
                    issue(m + 1, 0)
                drain_acc(1)
            return carry
        lax.fori_loop(0, n_mchunks, mchunk, 0)

        pltpu.sync_copy(acc_v, out_hbm.at[c, bb])

    return layer


_layer1 = _sc_sparse_layer(NNZ1, IN, HID)
_layer2 = _sc_sparse_layer(NNZ2, HID, OUT)


def _combine_sigmoid(parts, b):
    n = parts.shape[2]
    blk = 512

    def body(p_ref, b_ref, o_ref):
        o_ref[...] = jax.nn.sigmoid(p_ref[0] + p_ref[1] + b_ref[0][..., None])

    return pl.pallas_call(
        body,
        grid=(NB, n // blk),
        in_specs=[
            pl.BlockSpec((2, 1, blk, L), lambda i, j: (0, i, j, 0)),
            pl.BlockSpec((1, blk), lambda i, j: (0, j)),
        ],
        out_specs=pl.BlockSpec((1, blk, L), lambda i, j: (i, j, 0)),
        out_shape=jax.ShapeDtypeStruct((NB, n, L), jnp.float32),
    )(parts, b.reshape(1, -1)).reshape(NB * n, L)


def _finish(parts, b):
    def body(p_ref, b_ref, o_ref):
        o_ref[...] = p_ref[0] + p_ref[1] + b_ref[0][None, :, None]

    return pl.pallas_call(
        body,
        out_shape=jax.ShapeDtypeStruct((NB, OUT, L), jnp.float32),
    )(parts, b.reshape(1, -1))


@jax.jit
def kernel(x, connections1, connections2, w1, b1, w2, b2):
    xb = x.reshape(NB, L, IN).transpose(0, 2, 1).reshape(NB * IN, L)
    h_parts = _layer1(xb, connections1[0], connections1[1], w1)
    hb = _combine_sigmoid(h_parts, b1)
    o_parts = _layer2(hb, connections2[0], connections2[1], w2)
    ob = _finish(o_parts, b2)
    return ob.transpose(0, 2, 1).reshape(B, OUT)
